# baseline (speedup 1.0000x reference)
import jax
import jax.numpy as jnp
from jax.experimental import pallas as pl


def kernel(stims, atnTensor, atnLens, nameMap, embed, Wk, bk, Wv, bv):
    b, nAtn, nArgs, nAtnArg, keyDim = atnTensor.shape
    H = embed.shape[1]
    ev = jax.lax.dot_general(embed, Wv, (((1,), (0,)), ((), ())),
                             precision=jax.lax.Precision.HIGHEST) + 0.5 * bv
    flat = atnTensor.reshape(-1)
    names = jnp.take(nameMap, flat, axis=0)
    rows = jnp.take(ev, names, axis=0).reshape(b, nAtn, nArgs, nAtnArg, H)
    pre = rows.sum(-2)
    v = jax.nn.relu(pre)
    k = jax.nn.relu(jax.lax.dot_general(stims, Wk, (((1,), (0,)), ((), ())),
                                        precision=jax.lax.Precision.HIGHEST) + bk)
    x = jnp.einsum("bh,banh->ban", k, v,
                   precision=jax.lax.Precision.HIGHEST)
    inds = jnp.arange(nArgs)[None, None, :]
    mask = inds < atnLens[:, :, None]
    masked = jnp.where(mask, x, -1e9)
    xIdx = jnp.argmax(masked, axis=-1)
    return (x, xIdx)


# speedup vs baseline: 2.6516x; 2.6516x over previous
import functools

import jax
import jax.numpy as jnp
from jax import lax
from jax.experimental import pallas as pl
from jax.experimental.pallas import tpu as pltpu
from jax.experimental.pallas import tpu_sc as plsc

B, NATN, NARGS, NATNARG = 8, 8, 256, 2
H = 512
NKEYS = 8192
NEMBED = 4096

NITEMS = B * NATN * NARGS
NW = 32
ITEMS_PER_W = NITEMS // NW
KEYS_PER_W = ITEMS_PER_W * NATNARG
CH = 64
NCHUNK = ITEMS_PER_W // CH


def _sc_gather_kernel(keys_hbm, nmap_hbm, embed_hbm, targs_hbm,
                      nm_v, keys_v, idx_v, rows_v, out_v, sem):
    nc = 2
    wid = lax.axis_index("s") * nc + lax.axis_index("c")
    kbase = wid * KEYS_PER_W
    ibase = wid * ITEMS_PER_W

    pltpu.sync_copy(nmap_hbm, nm_v)
    pltpu.sync_copy(keys_hbm.at[pl.ds(kbase, KEYS_PER_W)], keys_v)

    def name_body(i, _):
        k16 = keys_v[pl.ds(i * 16, 16)]
        idx_v[pl.ds(i * 16, 16)] = plsc.load_gather(nm_v, [k16])
        return 0

    lax.fori_loop(0, KEYS_PER_W // 16, name_body, 0)

    for g in range(NCHUNK):
        pltpu.async_copy(
            embed_hbm.at[idx_v.at[pl.ds(g * 2 * CH, 2 * CH)]],
            rows_v, sem).wait()

        def add_body(i, _):
            for hc in range(H // 16):
                s = pl.ds(hc * 16, 16)
                out_v[i, s] = rows_v[2 * i, s] + rows_v[2 * i + 1, s]
            return 0

        lax.fori_loop(0, CH, add_body, 0)
        pltpu.sync_copy(out_v, targs_hbm.at[pl.ds(ibase + g * CH, CH)])


def _sc_gather(keys, nameMap, embed):
    mesh = plsc.VectorSubcoreMesh(core_axis_name="c", subcore_axis_name="s")
    fn = functools.partial(
        pl.kernel,
        mesh=mesh,
        out_type=jax.ShapeDtypeStruct((NITEMS, H), jnp.float32),
        scratch_types=[
            pltpu.VMEM((NKEYS,), jnp.int32),
            pltpu.VMEM((KEYS_PER_W,), jnp.int32),
            pltpu.VMEM((KEYS_PER_W,), jnp.int32),
            pltpu.VMEM((2 * CH, H), jnp.float32),
            pltpu.VMEM((CH, H), jnp.float32),
            pltpu.SemaphoreType.DMA,
        ],
        compiler_params=pltpu.CompilerParams(needs_layout_passes=False),
    )(_sc_gather_kernel)
    return fn(keys, nameMap, embed)


def _tc_kernel(lens_ref, targs_ref, wv_ref, bv_ref, stims_ref, wk_ref,
               bk_ref, x_ref, idx_ref, k_scr):
    step = pl.program_id(0)

    @pl.when(step == 0)
    def _():
        k_scr[...] = jnp.maximum(
            jax.lax.dot_general(stims_ref[...], wk_ref[...],
                                (((1,), (0,)), ((), ()))) + bk_ref[...], 0.0)

    b = step // NATN
    a = step % NATN

    v = jnp.maximum(
        jax.lax.dot_general(targs_ref[0], wv_ref[...],
                            (((1,), (0,)), ((), ()))) + bv_ref[...], 0.0)
    kb = k_scr[pl.ds(b, 1), :]
    xrow = jax.lax.dot_general(kb, v, (((1,), (1,)), ((), ())),
                               precision=jax.lax.Precision.HIGHEST)
    x_ref[0] = xrow

    ln = lens_ref[b, a]
    ids = lax.broadcasted_iota(jnp.int32, (1, NARGS), 1)
    masked = jnp.where(ids < ln, xrow, -1e9)
    xmax = jnp.max(masked)
    sel = jnp.where(masked == xmax, ids, NARGS)
    amin = jnp.min(sel)
    idx_ref[0, 0, :] = jnp.broadcast_to(amin, (128,))


def _tc_stage(targs, Wv, bv, stims, Wk, bk, atnLens):
    grid = (B * NATN,)
    x, idx = pl.pallas_call(
        _tc_kernel,
        grid=grid,
        in_specs=[
            pl.BlockSpec(memory_space=pltpu.SMEM),
            pl.BlockSpec((1, NARGS, H), lambda i: (i, 0, 0)),
            pl.BlockSpec((H, H), lambda i: (0, 0)),
            pl.BlockSpec((1, H), lambda i: (0, 0)),
            pl.BlockSpec((B, H), lambda i: (0, 0)),
            pl.BlockSpec((H, H), lambda i: (0, 0)),
            pl.BlockSpec((1, H), lambda i: (0, 0)),
        ],
        out_specs=[
            pl.BlockSpec((1, 1, NARGS), lambda i: (i, 0, 0)),
            pl.BlockSpec((1, 1, 128), lambda i: (i, 0, 0)),
        ],
        out_shape=[
            jax.ShapeDtypeStruct((B * NATN, 1, NARGS), jnp.float32),
            jax.ShapeDtypeStruct((B * NATN, 1, 128), jnp.int32),
        ],
        scratch_shapes=[pltpu.VMEM((B, H), jnp.float32)],
    )(atnLens, targs.reshape(B * NATN, NARGS, H), Wv, bv, stims, Wk, bk)
    return x, idx


def kernel(stims, atnTensor, atnLens, nameMap, embed, Wk, bk, Wv, bv):
    keys = atnTensor.reshape(-1).astype(jnp.int32)
    targs = _sc_gather(keys, nameMap.astype(jnp.int32), embed)
    x, idx = _tc_stage(targs, Wv, bv.reshape(1, H), stims, Wk,
                       bk.reshape(1, H), atnLens)
    xIdx = idx[:, 0, 0].reshape(B, NATN).astype(jnp.int32)
    return (x.reshape(B, NATN, NARGS), xIdx)


# speedup vs baseline: 3.1410x; 1.1846x over previous
import functools

import jax
import jax.numpy as jnp
from jax import lax
from jax.experimental import pallas as pl
from jax.experimental.pallas import tpu as pltpu
from jax.experimental.pallas import tpu_sc as plsc

B, NATN, NARGS, NATNARG = 8, 8, 256, 2
H = 512
NKEYS = 8192
NEMBED = 4096

NITEMS = B * NATN * NARGS
NW = 32
ITEMS_PER_W = NITEMS // NW
KEYS_PER_W = ITEMS_PER_W * NATNARG
CH = 64
NCHUNK = ITEMS_PER_W // CH


def _sc_gather_kernel(keys_hbm, nmap_hbm, embed_hbm, targs_hbm,
                      nm_v, keys_v, idx_v, rows_v, out_v, sem):
    nc = 2
    wid = lax.axis_index("s") * nc + lax.axis_index("c")
    kbase = wid * KEYS_PER_W
    ibase = wid * ITEMS_PER_W

    pltpu.sync_copy(nmap_hbm, nm_v)
    pltpu.sync_copy(keys_hbm.at[pl.ds(kbase, KEYS_PER_W)], keys_v)

    def name_body(i, _):
        k16 = keys_v[pl.ds(i * 16, 16)]
        idx_v[pl.ds(i * 16, 16)] = plsc.load_gather(nm_v, [k16])
        return 0

    lax.fori_loop(0, KEYS_PER_W // 16, name_body, 0)

    for g in range(NCHUNK):
        pltpu.async_copy(
            embed_hbm.at[idx_v.at[pl.ds(g * 2 * CH, 2 * CH)]],
            rows_v, sem).wait()

        def add_body(i, _):
            for hc in range(H // 16):
                s = pl.ds(hc * 16, 16)
                out_v[i, s] = rows_v[2 * i, s] + rows_v[2 * i + 1, s]
            return 0

        lax.fori_loop(0, CH, add_body, 0)
        pltpu.sync_copy(out_v, targs_hbm.at[pl.ds(ibase + g * CH, CH)])


def _sc_gather(keys, nameMap, embed):
    mesh = plsc.VectorSubcoreMesh(core_axis_name="c", subcore_axis_name="s")
    fn = functools.partial(
        pl.kernel,
        mesh=mesh,
        out_type=jax.ShapeDtypeStruct((NITEMS, H), jnp.float32),
        scratch_types=[
            pltpu.VMEM((NKEYS,), jnp.int32),
            pltpu.VMEM((KEYS_PER_W,), jnp.int32),
            pltpu.VMEM((KEYS_PER_W,), jnp.int32),
            pltpu.VMEM((2 * CH, H), jnp.float32),
            pltpu.VMEM((CH, H), jnp.float32),
            pltpu.SemaphoreType.DMA,
        ],
        compiler_params=pltpu.CompilerParams(needs_layout_passes=False),
    )(_sc_gather_kernel)
    return fn(keys, nameMap, embed)


def _tc_kernel(lens_ref, targs_ref, wv_ref, bv_ref, stims_ref, wk_ref,
               bk_ref, x_ref, idx_ref, k_scr):
    b = pl.program_id(0)

    @pl.when(b == 0)
    def _():
        k_scr[...] = jnp.maximum(
            jax.lax.dot_general(stims_ref[...], wk_ref[...],
                                (((1,), (0,)), ((), ()))) + bk_ref[...], 0.0)

    v = jnp.maximum(
        jax.lax.dot_general(targs_ref[0], wv_ref[...],
                            (((1,), (0,)), ((), ()))) + bv_ref[...], 0.0)
    kb = k_scr[pl.ds(b, 1), :]
    xrow = jax.lax.dot_general(kb, v, (((1,), (1,)), ((), ())),
                               precision=jax.lax.Precision.HIGHEST)
    x_ref[0] = xrow

    ids = lax.broadcasted_iota(jnp.int32, (1, NARGS), 1)
    out = jnp.zeros((1, 128), jnp.int32)
    lane = lax.broadcasted_iota(jnp.int32, (1, 128), 1)
    for a in range(NATN):
        xa = xrow[:, a * NARGS:(a + 1) * NARGS]
        masked = jnp.where(ids < lens_ref[b, a], xa, -1e9)
        xmax = jnp.max(masked)
        amin = jnp.min(jnp.where(masked == xmax, ids, NARGS))
        out = jnp.where(lane == a, amin, out)
    idx_ref[0] = out


def _tc_stage(targs, Wv, bv, stims, Wk, bk, atnLens):
    x, idx = pl.pallas_call(
        _tc_kernel,
        grid=(B,),
        in_specs=[
            pl.BlockSpec(memory_space=pltpu.SMEM),
            pl.BlockSpec((1, NATN * NARGS, H), lambda i: (i, 0, 0)),
            pl.BlockSpec((H, H), lambda i: (0, 0)),
            pl.BlockSpec((1, H), lambda i: (0, 0)),
            pl.BlockSpec((B, H), lambda i: (0, 0)),
            pl.BlockSpec((H, H), lambda i: (0, 0)),
            pl.BlockSpec((1, H), lambda i: (0, 0)),
        ],
        out_specs=[
            pl.BlockSpec((1, 1, NATN * NARGS), lambda i: (i, 0, 0)),
            pl.BlockSpec((1, 1, 128), lambda i: (i, 0, 0)),
        ],
        out_shape=[
            jax.ShapeDtypeStruct((B, 1, NATN * NARGS), jnp.float32),
            jax.ShapeDtypeStruct((B, 1, 128), jnp.int32),
        ],
        scratch_shapes=[pltpu.VMEM((B, H), jnp.float32)],
    )(atnLens, targs.reshape(B, NATN * NARGS, H), Wv, bv, stims, Wk, bk)
    return x, idx


def kernel(stims, atnTensor, atnLens, nameMap, embed, Wk, bk, Wv, bv):
    keys = atnTensor.reshape(-1).astype(jnp.int32)
    targs = _sc_gather(keys, nameMap.astype(jnp.int32), embed)
    x, idx = _tc_stage(targs, Wv, bv.reshape(1, H), stims, Wk,
                       bk.reshape(1, H), atnLens)
    xIdx = idx[:, 0, :NATN].astype(jnp.int32)
    return (x.reshape(B, NATN, NARGS), xIdx)


# speedup vs baseline: 5.1511x; 1.6400x over previous
import functools

import jax
import jax.numpy as jnp
from jax import lax
from jax.experimental import pallas as pl
from jax.experimental.pallas import tpu as pltpu
from jax.experimental.pallas import tpu_sc as plsc

B, NATN, NARGS, NATNARG = 8, 8, 256, 2
H = 512
NKEYS = 8192
NEMBED = 4096

NITEMS = B * NATN * NARGS
NW = 32
ITEMS_PER_W = NITEMS // NW
KEYS_PER_W = ITEMS_PER_W * NATNARG
CH = 32
NCHUNK = ITEMS_PER_W // CH


def _sc_gather_kernel(keys_hbm, nmap_hbm, embed_hbm, targs_hbm,
                      nm_v, keys_v, idx_e, idx_o,
                      rowe0_v, rowe1_v, rowo0_v, rowo1_v, out0_v, out1_v,
                      gsem0, gsem1, wsem0, wsem1):
    nc = 2
    wid = lax.axis_index("s") * nc + lax.axis_index("c")
    kbase = wid * KEYS_PER_W
    ibase = wid * ITEMS_PER_W

    pltpu.sync_copy(nmap_hbm, nm_v)
    pltpu.sync_copy(keys_hbm.at[pl.ds(kbase, KEYS_PER_W)], keys_v)

    lanes = lax.iota(jnp.int32, 16)

    def name_body(i, _):
        base = i * 32
        ke = plsc.load_gather(keys_v, [base + 2 * lanes])
        ko = plsc.load_gather(keys_v, [base + 2 * lanes + 1])
        idx_e[pl.ds(i * 16, 16)] = plsc.load_gather(nm_v, [ke])
        idx_o[pl.ds(i * 16, 16)] = plsc.load_gather(nm_v, [ko])
        return 0

    lax.fori_loop(0, KEYS_PER_W // 32, name_body, 0)

    ebufs = (rowe0_v, rowe1_v)
    obufs = (rowo0_v, rowo1_v)
    outs = (out0_v, out1_v)
    gsems = (gsem0, gsem1)
    wsems = (wsem0, wsem1)

    def start_gathers(g, p):
        de = pltpu.async_copy(
            embed_hbm.at[idx_e.at[pl.ds(g * CH, CH)]], ebufs[p], gsems[p])
        do = pltpu.async_copy(
            embed_hbm.at[idx_o.at[pl.ds(g * CH, CH)]], obufs[p], gsems[p])
        return (de, do)

    gathers = [None, None]
    writes = [None, None]
    gathers[0] = start_gathers(0, 0)
    for g in range(NCHUNK):
        p = g % 2
        de, do = gathers[p]
        de.wait()
        do.wait()
        if g + 1 < NCHUNK:
            gathers[1 - p] = start_gathers(g + 1, 1 - p)
        if writes[p] is not None:
            writes[p].wait()
        be, bo, out = ebufs[p], obufs[p], outs[p]

        def add_body(i, _):
            for hc in range(H // 16):
                s = pl.ds(hc * 16, 16)
                out[i, s] = be[i, s] + bo[i, s]
            return 0

        lax.fori_loop(0, CH, add_body, 0)
        w = pltpu.make_async_copy(
            out, targs_hbm.at[pl.ds(ibase + g * CH, CH)], wsems[p])
        w.start()
        writes[p] = w
    writes[0].wait()
    writes[1].wait()


def _sc_gather(keys, nameMap, embed):
    mesh = plsc.VectorSubcoreMesh(core_axis_name="c", subcore_axis_name="s")
    fn = functools.partial(
        pl.kernel,
        mesh=mesh,
        out_type=jax.ShapeDtypeStruct((NITEMS, H), jnp.float32),
        scratch_types=[
            pltpu.VMEM((NKEYS,), jnp.int32),
            pltpu.VMEM((KEYS_PER_W,), jnp.int32),
            pltpu.VMEM((ITEMS_PER_W,), jnp.int32),
            pltpu.VMEM((ITEMS_PER_W,), jnp.int32),
            pltpu.VMEM((CH, H), jnp.float32),
            pltpu.VMEM((CH, H), jnp.float32),
            pltpu.VMEM((CH, H), jnp.float32),
            pltpu.VMEM((CH, H), jnp.float32),
            pltpu.VMEM((CH, H), jnp.float32),
            pltpu.VMEM((CH, H), jnp.float32),
            pltpu.SemaphoreType.DMA,
            pltpu.SemaphoreType.DMA,
            pltpu.SemaphoreType.DMA,
            pltpu.SemaphoreType.DMA,
        ],
        compiler_params=pltpu.CompilerParams(needs_layout_passes=False),
    )(_sc_gather_kernel)
    return fn(keys, nameMap, embed)


def _tc_kernel(lens_ref, targs_ref, wv_ref, bv_ref, stims_ref, wk_ref,
               bk_ref, x_ref, idx_ref, k_scr):
    b = pl.program_id(0)

    @pl.when(b == 0)
    def _():
        k_scr[...] = jnp.maximum(
            jax.lax.dot_general(stims_ref[...], wk_ref[...],
                                (((1,), (0,)), ((), ()))) + bk_ref[...], 0.0)

    v = jnp.maximum(
        jax.lax.dot_general(targs_ref[0], wv_ref[...],
                            (((1,), (0,)), ((), ()))) + bv_ref[...], 0.0)
    kb = k_scr[pl.ds(b, 1), :]
    xrow = jax.lax.dot_general(kb, v, (((1,), (1,)), ((), ())),
                               precision=jax.lax.Precision.HIGHEST)
    x_ref[0] = xrow

    ids = lax.broadcasted_iota(jnp.int32, (1, NARGS), 1)
    out = jnp.zeros((1, 128), jnp.int32)
    lane = lax.broadcasted_iota(jnp.int32, (1, 128), 1)
    for a in range(NATN):
        xa = xrow[:, a * NARGS:(a + 1) * NARGS]
        masked = jnp.where(ids < lens_ref[b, a], xa, -1e9)
        xmax = jnp.max(masked)
        amin = jnp.min(jnp.where(masked == xmax, ids, NARGS))
        out = jnp.where(lane == a, amin, out)
    idx_ref[0] = out


def _tc_stage(targs, Wv, bv, stims, Wk, bk, atnLens):
    x, idx = pl.pallas_call(
        _tc_kernel,
        grid=(B,),
        in_specs=[
            pl.BlockSpec(memory_space=pltpu.SMEM),
            pl.BlockSpec((1, NATN * NARGS, H), lambda i: (i, 0, 0)),
            pl.BlockSpec((H, H), lambda i: (0, 0)),
            pl.BlockSpec((1, H), lambda i: (0, 0)),
            pl.BlockSpec((B, H), lambda i: (0, 0)),
            pl.BlockSpec((H, H), lambda i: (0, 0)),
            pl.BlockSpec((1, H), lambda i: (0, 0)),
        ],
        out_specs=[
            pl.BlockSpec((1, 1, NATN * NARGS), lambda i: (i, 0, 0)),
            pl.BlockSpec((1, 1, 128), lambda i: (i, 0, 0)),
        ],
        out_shape=[
            jax.ShapeDtypeStruct((B, 1, NATN * NARGS), jnp.float32),
            jax.ShapeDtypeStruct((B, 1, 128), jnp.int32),
        ],
        scratch_shapes=[pltpu.VMEM((B, H), jnp.float32)],
    )(atnLens, targs.reshape(B, NATN * NARGS, H), Wv, bv, stims, Wk, bk)
    return x, idx


def kernel(stims, atnTensor, atnLens, nameMap, embed, Wk, bk, Wv, bv):
    keys = atnTensor.reshape(-1).astype(jnp.int32)
    targs = _sc_gather(keys, nameMap.astype(jnp.int32), embed)
    x, idx = _tc_stage(targs, Wv, bv.reshape(1, H), stims, Wk,
                       bk.reshape(1, H), atnLens)
    xIdx = idx[:, 0, :NATN].astype(jnp.int32)
    return (x.reshape(B, NATN, NARGS), xIdx)


# speedup vs baseline: 5.4206x; 1.0523x over previous
import functools

import jax
import jax.numpy as jnp
from jax import lax
from jax.experimental import pallas as pl
from jax.experimental.pallas import tpu as pltpu
from jax.experimental.pallas import tpu_sc as plsc

B, NATN, NARGS, NATNARG = 8, 8, 256, 2
H = 512
NKEYS = 8192

NHALF = 2
BH = B // NHALF
NITEMS_H = BH * NATN * NARGS
NW = 32
IPW = NITEMS_H // NW
KPW = IPW * NATNARG
CH = 32
NCHUNK = IPW // CH


def _sc_gather_kernel(half, keys_hbm, nmap_hbm, embed_hbm, targs_hbm,
                      nm_v, keys_v, idx_e, idx_o,
                      rowe0_v, rowe1_v, rowo0_v, rowo1_v, out0_v, out1_v,
                      gsem0, gsem1, wsem0, wsem1):
    nc = 2
    wid = lax.axis_index("s") * nc + lax.axis_index("c")
    kbase = half * NITEMS_H * NATNARG + wid * KPW
    ibase = wid * IPW

    pltpu.sync_copy(nmap_hbm, nm_v)
    pltpu.sync_copy(keys_hbm.at[pl.ds(kbase, KPW)], keys_v)

    lanes = lax.iota(jnp.int32, 16)

    def name_body(i, _):
        base = i * 32
        ke = plsc.load_gather(keys_v, [base + 2 * lanes])
        ko = plsc.load_gather(keys_v, [base + 2 * lanes + 1])
        idx_e[pl.ds(i * 16, 16)] = plsc.load_gather(nm_v, [ke])
        idx_o[pl.ds(i * 16, 16)] = plsc.load_gather(nm_v, [ko])
        return 0

    lax.fori_loop(0, KPW // 32, name_body, 0)

    ebufs = (rowe0_v, rowe1_v)
    obufs = (rowo0_v, rowo1_v)
    outs = (out0_v, out1_v)
    gsems = (gsem0, gsem1)
    wsems = (wsem0, wsem1)

    def start_gathers(g, p):
        de = pltpu.async_copy(
            embed_hbm.at[idx_e.at[pl.ds(g * CH, CH)]], ebufs[p], gsems[p])
        do = pltpu.async_copy(
            embed_hbm.at[idx_o.at[pl.ds(g * CH, CH)]], obufs[p], gsems[p])
        return (de, do)

    gathers = [None, None]
    writes = [None, None]
    gathers[0] = start_gathers(0, 0)
    for g in range(NCHUNK):
        p = g % 2
        de, do = gathers[p]
        de.wait()
        do.wait()
        if g + 1 < NCHUNK:
            gathers[1 - p] = start_gathers(g + 1, 1 - p)
        if writes[p] is not None:
            writes[p].wait()
        be, bo, out = ebufs[p], obufs[p], outs[p]

        def add_body(i, _):
            for hc in range(H // 16):
                s = pl.ds(hc * 16, 16)
                out[i, s] = be[i, s] + bo[i, s]
            return 0

        lax.fori_loop(0, CH, add_body, 0)
        w = pltpu.make_async_copy(
            out, targs_hbm.at[pl.ds(ibase + g * CH, CH)], wsems[p])
        w.start()
        writes[p] = w
    writes[0].wait()
    writes[1].wait()


def _sc_gather(keys, nameMap, embed, half):
    mesh = plsc.VectorSubcoreMesh(core_axis_name="c", subcore_axis_name="s")
    fn = functools.partial(
        pl.kernel,
        mesh=mesh,
        out_type=jax.ShapeDtypeStruct((NITEMS_H, H), jnp.float32),
        scratch_types=[
            pltpu.VMEM((NKEYS,), jnp.int32),
            pltpu.VMEM((KPW,), jnp.int32),
            pltpu.VMEM((IPW,), jnp.int32),
            pltpu.VMEM((IPW,), jnp.int32),
            pltpu.VMEM((CH, H), jnp.float32),
            pltpu.VMEM((CH, H), jnp.float32),
            pltpu.VMEM((CH, H), jnp.float32),
            pltpu.VMEM((CH, H), jnp.float32),
            pltpu.VMEM((CH, H), jnp.float32),
            pltpu.VMEM((CH, H), jnp.float32),
            pltpu.SemaphoreType.DMA,
            pltpu.SemaphoreType.DMA,
            pltpu.SemaphoreType.DMA,
            pltpu.SemaphoreType.DMA,
        ],
        compiler_params=pltpu.CompilerParams(needs_layout_passes=False),
    )(functools.partial(_sc_gather_kernel, half))
    return fn(keys, nameMap, embed)


def _tc_kernel(half, lens_ref, targs_ref, wv_ref, bv_ref, stims_ref, wk_ref,
               bk_ref, x_ref, idx_ref, k_scr):
    b = pl.program_id(0)
    bg = half * BH + b

    @pl.when(b == 0)
    def _():
        k_scr[...] = jnp.maximum(
            jax.lax.dot_general(stims_ref[...], wk_ref[...],
                                (((1,), (0,)), ((), ()))) + bk_ref[...], 0.0)

    v = jnp.maximum(
        jax.lax.dot_general(targs_ref[0], wv_ref[...],
                            (((1,), (0,)), ((), ()))) + bv_ref[...], 0.0)
    kb = k_scr[pl.ds(bg, 1), :]
    xrow = jax.lax.dot_general(kb, v, (((1,), (1,)), ((), ())),
                               precision=jax.lax.Precision.HIGHEST)
    x_ref[0] = xrow

    ids = lax.broadcasted_iota(jnp.int32, (1, NARGS), 1)
    out = jnp.zeros((1, 128), jnp.int32)
    lane = lax.broadcasted_iota(jnp.int32, (1, 128), 1)
    for a in range(NATN):
        xa = xrow[:, a * NARGS:(a + 1) * NARGS]
        masked = jnp.where(ids < lens_ref[bg, a], xa, -1e9)
        xmax = jnp.max(masked)
        amin = jnp.min(jnp.where(masked == xmax, ids, NARGS))
        out = jnp.where(lane == a, amin, out)
    idx_ref[0] = out


def _tc_stage(targs, Wv, bv, stims, Wk, bk, atnLens, half):
    x, idx = pl.pallas_call(
        functools.partial(_tc_kernel, half),
        grid=(BH,),
        in_specs=[
            pl.BlockSpec(memory_space=pltpu.SMEM),
            pl.BlockSpec((1, NATN * NARGS, H), lambda i: (i, 0, 0)),
            pl.BlockSpec((H, H), lambda i: (0, 0)),
            pl.BlockSpec((1, H), lambda i: (0, 0)),
            pl.BlockSpec((B, H), lambda i: (0, 0)),
            pl.BlockSpec((H, H), lambda i: (0, 0)),
            pl.BlockSpec((1, H), lambda i: (0, 0)),
        ],
        out_specs=[
            pl.BlockSpec((1, 1, NATN * NARGS), lambda i: (i, 0, 0)),
            pl.BlockSpec((1, 1, 128), lambda i: (i, 0, 0)),
        ],
        out_shape=[
            jax.ShapeDtypeStruct((BH, 1, NATN * NARGS), jnp.float32),
            jax.ShapeDtypeStruct((BH, 1, 128), jnp.int32),
        ],
        scratch_shapes=[pltpu.VMEM((B, H), jnp.float32)],
    )(atnLens, targs.reshape(BH, NATN * NARGS, H), Wv, bv, stims, Wk, bk)
    return x, idx


def kernel(stims, atnTensor, atnLens, nameMap, embed, Wk, bk, Wv, bv):
    keys = atnTensor.reshape(-1).astype(jnp.int32)
    nmap = nameMap.astype(jnp.int32)
    bv2 = bv.reshape(1, H)
    bk2 = bk.reshape(1, H)
    xs, idxs = [], []
    for half in range(NHALF):
        targs = _sc_gather(keys, nmap, embed, half)
        x, idx = _tc_stage(targs, Wv, bv2, stims, Wk, bk2, atnLens, half)
        xs.append(x)
        idxs.append(idx)
    x = jnp.concatenate(xs, axis=0)
    idx = jnp.concatenate(idxs, axis=0)
    xIdx = idx[:, 0, :NATN].astype(jnp.int32)
    return (x.reshape(B, NATN, NARGS), xIdx)
